# baseline (speedup 1.0000x reference)
import functools

import jax
import jax.numpy as jnp
from jax import lax
from jax.experimental import pallas as pl
from jax.experimental.pallas import tpu as pltpu
from jax.experimental.pallas import tpu_sc as plsc

DIM = 64
MARGIN = 0.1
B = 4096
_L = 16

_INFO = plsc.get_sparse_core_info()
_NC = _INFO.num_cores
_NS = _INFO.num_subcores
_NW = _NC * _NS
_RPW = B // _NW
_C = 64
_NCHUNK = _RPW // _C


def _sc_body(nf1a_h, nf1b_h, nf2a_h, nf2b_h, nf2c_h, nf3a_h, nf3b_h, nf3c_h,
             ce_h, bu_h, rh_h, rt_h,
             s1_h, p2_h, q2_h, e1_h, e2_h,
             i1a, i1b, i2a, i2b, i2c, i3a, i3b, i3c, i3ah, i3ch,
             g1a, g1b, g2a, g2b, g2c, g3a, g3b, gba, gbb, grh, grt,
             o1, o2p, o2q, o3a, o3b, sem):
    wid = lax.axis_index("s") * _NC + lax.axis_index("c")

    for chunk in range(_NCHUNK):
        base = wid * _RPW + chunk * _C
        sl = pl.ds(base, _C)
        pltpu.sync_copy(nf1a_h.at[sl], i1a)
        pltpu.sync_copy(nf1b_h.at[sl], i1b)
        pltpu.sync_copy(nf2a_h.at[sl], i2a)
        pltpu.sync_copy(nf2b_h.at[sl], i2b)
        pltpu.sync_copy(nf2c_h.at[sl], i2c)
        pltpu.sync_copy(nf3a_h.at[sl], i3a)
        pltpu.sync_copy(nf3b_h.at[sl], i3b)
        pltpu.sync_copy(nf3c_h.at[sl], i3c)
        for k in range(_C // _L):
            ks = pl.ds(k * _L, _L)
            i3ah[ks] = lax.shift_right_logical(i3a[ks], 1)
            i3ch[ks] = lax.shift_right_logical(i3c[ks], 1)

        cps = [
            pltpu.async_copy(ce_h.at[i1a], g1a, sem),
            pltpu.async_copy(ce_h.at[i1b], g1b, sem),
            pltpu.async_copy(ce_h.at[i2a], g2a, sem),
            pltpu.async_copy(ce_h.at[i2b], g2b, sem),
            pltpu.async_copy(ce_h.at[i2c], g2c, sem),
            pltpu.async_copy(ce_h.at[i3a], g3a, sem),
            pltpu.async_copy(ce_h.at[i3c], g3b, sem),
            pltpu.async_copy(bu_h.at[i3ah], gba, sem),
            pltpu.async_copy(bu_h.at[i3ch], gbb, sem),
            pltpu.async_copy(rh_h.at[i3b], grh, sem),
            pltpu.async_copy(rt_h.at[i3b], grt, sem),
        ]
        for cp in cps:
            cp.wait()

        def row_body(i, carry):
            z = jnp.zeros((_L,), jnp.float32)
            acc1, accp, accq, accd1, accd2 = z, z, z, z, z
            gbase = lax.bitwise_and(i, jnp.int32(-_L))
            lane = lax.bitwise_and(i, jnp.int32(_L - 1))
            onehot = lax.broadcasted_iota(jnp.int32, (_L,), 0) == lane
            zi = jnp.zeros((_L,), jnp.int32)
            ra = jnp.sum(jnp.where(onehot, i3a[pl.ds(gbase, _L)], zi))
            rc = jnp.sum(jnp.where(onehot, i3c[pl.ds(gbase, _L)], zi))
            ma = jnp.full((_L,), lax.bitwise_and(ra, 1), jnp.int32) == 1
            mc = jnp.full((_L,), lax.bitwise_and(rc, 1), jnp.int32) == 1
            for s in range(DIM // _L):
                dc = pl.ds(s * _L, _L)
                do = pl.ds(DIM + s * _L, _L)
                c1 = g1a[i, dc]
                o1v = jnp.abs(g1a[i, do])
                c2 = g1b[i, dc]
                o2v = jnp.abs(g1b[i, do])
                t = jnp.maximum(jnp.abs(c1 - c2) + o1v - o2v - MARGIN, 0.0)
                acc1 = acc1 + t * t
                cc = g2a[i, dc]
                co = jnp.abs(g2a[i, do])
                dcv = g2b[i, dc]
                dov = jnp.abs(g2b[i, do])
                ec = g2c[i, dc]
                eo = jnp.abs(g2c[i, do])
                low = jnp.maximum(cc - co, dcv - dov)
                up = jnp.minimum(cc + co, dcv + dov)
                icv = (low + up) * 0.5
                iov = jnp.abs(up - low) * 0.5
                tp = jnp.maximum(jnp.abs(icv - ec) + iov - eo - MARGIN, 0.0)
                accp = accp + tp * tp
                tq = jnp.maximum(low - up, 0.0)
                accq = accq + tq * tq
                c3 = g3a[i, dc]
                c3o = jnp.abs(g3a[i, do])
                d3 = g3b[i, dc]
                d3o = jnp.abs(g3b[i, do])
                hc = grh[i, dc]
                ho = jnp.abs(grh[i, do])
                tc = grt[i, dc]
                to = jnp.abs(grt[i, do])
                cb = jnp.where(ma, gba[i, do], gba[i, dc])
                db = jnp.where(mc, gbb[i, do], gbb[i, dc])
                t1 = jnp.maximum(jnp.abs(c3 + db - hc) + c3o - ho - MARGIN, 0.0)
                accd1 = accd1 + t1 * t1
                t2 = jnp.maximum(jnp.abs(d3 + cb - tc) + d3o - to - MARGIN, 0.0)
                accd2 = accd2 + t2 * t2
            o1[i, :] = acc1
            o2p[i, :] = accp
            o2q[i, :] = accq
            o3a[i, :] = accd1
            o3b[i, :] = accd2
            return carry

        lax.fori_loop(0, _C, row_body, 0)

        pltpu.sync_copy(o1, s1_h.at[sl, :])
        pltpu.sync_copy(o2p, p2_h.at[sl, :])
        pltpu.sync_copy(o2q, q2_h.at[sl, :])
        pltpu.sync_copy(o3a, e1_h.at[sl, :])
        pltpu.sync_copy(o3b, e2_h.at[sl, :])


_sc_rowstats = functools.partial(
    pl.kernel,
    mesh=plsc.VectorSubcoreMesh(core_axis_name="c", subcore_axis_name="s"),
    out_type=[jax.ShapeDtypeStruct((B, _L), jnp.float32) for _ in range(5)],
    compiler_params=pltpu.CompilerParams(
        needs_layout_passes=False, use_tc_tiling_on_sc=False),
    scratch_types=(
        [pltpu.VMEM((_C,), jnp.int32) for _ in range(10)]
        + [pltpu.VMEM((_C, 2 * DIM), jnp.float32) for _ in range(11)]
        + [pltpu.VMEM((_C, _L), jnp.float32) for _ in range(5)]
        + [pltpu.SemaphoreType.DMA]
    ),
)(_sc_body)


def _finish_body(s1, p2, q2, e1, e2, out):
    inv = 1.0 / B
    l1 = jnp.sum(s1[...]) * inv
    p2r = jnp.sum(p2[...], axis=1)
    q2r = jnp.sum(q2[...], axis=1)
    l2 = (jnp.sum(p2r) + jnp.sum(q2r)) * inv \
        + 2.0 * (jnp.sum(jnp.sqrt(p2r)) * inv) * (jnp.sum(jnp.sqrt(q2r)) * inv)
    a = jnp.sqrt(jnp.sum(e1[...], axis=1))
    b = jnp.sqrt(jnp.sum(e2[...], axis=1))
    l3 = jnp.sum((a + b) * (a + b)) * (0.25 * inv)
    out[0, 0] = l1 + l2 + l3


_finish = pl.pallas_call(
    _finish_body,
    out_shape=jax.ShapeDtypeStruct((1, 1), jnp.float32),
    out_specs=pl.BlockSpec(memory_space=pltpu.MemorySpace.SMEM),
)


def kernel(nf1, nf2, nf3, class_embeds, bumps, relation_heads, relation_tails):
    s1, p2, q2, e1, e2 = _sc_rowstats(
        nf1[:, 0], nf1[:, 1],
        nf2[:, 0], nf2[:, 1], nf2[:, 2],
        nf3[:, 0], nf3[:, 1], nf3[:, 2],
        class_embeds, bumps.reshape(-1, 2 * DIM),
        relation_heads, relation_tails,
    )
    r = _finish(s1, p2, q2, e1, e2)
    return r[0, 0]
